# baseline (speedup 1.0000x reference)
import jax
import jax.numpy as jnp
from jax import lax
from jax.experimental import pallas as pl
from jax.experimental.pallas import tpu as pltpu
from jax.experimental.pallas import tpu_sc as plsc

ITEM_NUM = 100000
FACTORS = 128
BATCH = 16384
HIST = 50

NC = 2
NS = 16
NW = NC * NS

CHUNK = 128
TOTAL = BATCH * HIST
PER_W = TOTAL // NW
K = PER_W // CHUNK
NBUF = 4


def _emb_body(table_hbm, idx_hbm, out_hbm, idx_v, rows_v, *sems):
    gsems = sems[:NBUF]
    osems = sems[NBUF:]
    wid = lax.axis_index("s") * NC + lax.axis_index("c")

    pltpu.sync_copy(idx_hbm.at[wid], idx_v)

    def start_gather(t, b):
        return pltpu.async_copy(
            table_hbm.at[idx_v.at[t]], rows_v.at[b], gsems[b])

    def wait_gather(t, b):
        pltpu.make_async_copy(
            table_hbm.at[idx_v.at[t]], rows_v.at[b], gsems[b]).wait()

    def start_write(t, b):
        return pltpu.async_copy(
            rows_v.at[b], out_hbm.at[wid * K + t], osems[b])

    def wait_write(t, b):
        pltpu.make_async_copy(
            rows_v.at[b], out_hbm.at[wid * K + t], osems[b]).wait()

    for b in range(NBUF):
        start_gather(b, b)

    @pl.loop(0, K - NBUF, step=NBUF)
    def _main(t0):
        for b in range(NBUF):
            t = t0 + b
            wait_gather(t, b)
            start_write(t, b)
        for b in range(NBUF):
            t = t0 + b
            wait_write(t, b)
            start_gather(t + NBUF, b)

    for b in range(NBUF):
        t = K - NBUF + b
        wait_gather(t, b)
        start_write(t, b)
    for b in range(NBUF):
        t = K - NBUF + b
        wait_write(t, b)


@jax.jit
def kernel(data, ivectors):
    idx = data.astype(jnp.int32).reshape(NW, K, CHUNK)
    mesh = plsc.VectorSubcoreMesh(core_axis_name="c", subcore_axis_name="s")
    run = pl.kernel(
        _emb_body,
        out_type=jax.ShapeDtypeStruct((NW * K, CHUNK, FACTORS), jnp.float32),
        mesh=mesh,
        scratch_types=(
            [pltpu.VMEM((K, CHUNK), jnp.int32),
             pltpu.VMEM((NBUF, CHUNK, FACTORS), jnp.float32)]
            + [pltpu.SemaphoreType.DMA] * (2 * NBUF)
        ),
    )
    out = run(ivectors, idx)
    return out.reshape(BATCH, HIST, FACTORS)


# speedup vs baseline: 1.0030x; 1.0030x over previous
import jax
import jax.numpy as jnp
from jax import lax
from jax.experimental import pallas as pl
from jax.experimental.pallas import tpu as pltpu
from jax.experimental.pallas import tpu_sc as plsc

ITEM_NUM = 100000
FACTORS = 128
BATCH = 16384
HIST = 50

NC = 2
NS = 16
NW = NC * NS

CHUNK = 128
TOTAL = BATCH * HIST
PER_W = TOTAL // NW
K = PER_W // CHUNK
NBUF = 5


def _emb_body(table_hbm, idx_hbm, out_hbm, idx_v, rows_v, *sems):
    gsems = sems[:NBUF]
    osems = sems[NBUF:]
    wid = lax.axis_index("s") * NC + lax.axis_index("c")

    pltpu.sync_copy(idx_hbm.at[wid], idx_v)

    def start_gather(t, b):
        return pltpu.async_copy(
            table_hbm.at[idx_v.at[t]], rows_v.at[b], gsems[b])

    def wait_gather(t, b):
        pltpu.make_async_copy(
            table_hbm.at[idx_v.at[t]], rows_v.at[b], gsems[b]).wait()

    def start_write(t, b):
        return pltpu.async_copy(
            rows_v.at[b], out_hbm.at[wid * K + t], osems[b])

    def wait_write(t, b):
        pltpu.make_async_copy(
            rows_v.at[b], out_hbm.at[wid * K + t], osems[b]).wait()

    for b in range(NBUF):
        start_gather(b, b)

    @pl.loop(0, K - NBUF, step=NBUF)
    def _main(t0):
        for b in range(NBUF):
            t = t0 + b
            wait_gather(t, b)
            start_write(t, b)
            wait_write(t, b)
            start_gather(t + NBUF, b)

    for b in range(NBUF):
        t = K - NBUF + b
        wait_gather(t, b)
        start_write(t, b)
    for b in range(NBUF):
        t = K - NBUF + b
        wait_write(t, b)


@jax.jit
def kernel(data, ivectors):
    idx = data.astype(jnp.int32).reshape(NW, K, CHUNK)
    mesh = plsc.VectorSubcoreMesh(core_axis_name="c", subcore_axis_name="s")
    run = pl.kernel(
        _emb_body,
        out_type=jax.ShapeDtypeStruct((NW * K, CHUNK, FACTORS), jnp.float32),
        mesh=mesh,
        scratch_types=(
            [pltpu.VMEM((K, CHUNK), jnp.int32),
             pltpu.VMEM((NBUF, CHUNK, FACTORS), jnp.float32)]
            + [pltpu.SemaphoreType.DMA] * (2 * NBUF)
        ),
    )
    out = run(ivectors, idx)
    return out.reshape(BATCH, HIST, FACTORS)


# speedup vs baseline: 3.4666x; 3.4562x over previous
import jax
import jax.numpy as jnp
from jax import lax
from jax.experimental import pallas as pl
from jax.experimental.pallas import tpu as pltpu
from jax.experimental.pallas import tpu_sc as plsc

ITEM_NUM = 100000
FACTORS = 128
BATCH = 16384
HIST = 50

NC = 2
NS = 16
NW = NC * NS

CHUNK = 128
TOTAL = BATCH * HIST
PER_W = TOTAL // NW
K = PER_W // CHUNK
NBUF = 5
LAG = 2


def _emb_body(table_hbm, idx_hbm, out_hbm, idx_v, rows_v, *sems):
    gsems = sems[:NBUF]
    osems = sems[NBUF:]
    wid = lax.axis_index("s") * NC + lax.axis_index("c")

    pltpu.sync_copy(idx_hbm.at[wid], idx_v)

    def start_gather(t, b):
        pltpu.async_copy(table_hbm.at[idx_v.at[t]], rows_v.at[b], gsems[b])

    def wait_gather(t, b):
        pltpu.make_async_copy(
            table_hbm.at[idx_v.at[t]], rows_v.at[b], gsems[b]).wait()

    def start_write(t, b):
        pltpu.async_copy(rows_v.at[b], out_hbm.at[wid * K + t], osems[b])

    def wait_write(t, b):
        pltpu.make_async_copy(
            rows_v.at[b], out_hbm.at[wid * K + t], osems[b]).wait()

    for b in range(NBUF):
        start_gather(b, b)

    for t in range(LAG):
        wait_gather(t, t % NBUF)
        start_write(t, t % NBUF)

    @pl.loop(LAG, K - NBUF + LAG, step=NBUF)
    def _main(t0):
        for i in range(NBUF):
            t = t0 + i
            b = (LAG + i) % NBUF
            bl = i % NBUF
            wait_gather(t, b)
            start_write(t, b)
            wait_write(t - LAG, bl)
            start_gather(t - LAG + NBUF, bl)

    for t in range(K - NBUF + LAG, K):
        wait_gather(t, t % NBUF)
        start_write(t, t % NBUF)
    for t in range(K - NBUF, K):
        wait_write(t, t % NBUF)


@jax.jit
def kernel(data, ivectors):
    idx = jnp.transpose(data.astype(jnp.int32)).reshape(NW, K, CHUNK)
    mesh = plsc.VectorSubcoreMesh(core_axis_name="c", subcore_axis_name="s")
    run = pl.kernel(
        _emb_body,
        out_type=jax.ShapeDtypeStruct((NW * K, CHUNK, FACTORS), jnp.float32),
        mesh=mesh,
        scratch_types=(
            [pltpu.VMEM((K, CHUNK), jnp.int32),
             pltpu.VMEM((NBUF, CHUNK, FACTORS), jnp.float32)]
            + [pltpu.SemaphoreType.DMA] * (2 * NBUF)
        ),
    )
    out = run(ivectors, idx)
    return jnp.transpose(out.reshape(HIST, BATCH, FACTORS), (1, 0, 2))


# speedup vs baseline: 3.4746x; 1.0023x over previous
import jax
import jax.numpy as jnp
from jax import lax
from jax.experimental import pallas as pl
from jax.experimental.pallas import tpu as pltpu
from jax.experimental.pallas import tpu_sc as plsc

ITEM_NUM = 100000
FACTORS = 128
BATCH = 16384
HIST = 50

NC = 2
NS = 16
NW = NC * NS

CHUNK = 128
TOTAL = BATCH * HIST
PER_W = TOTAL // NW
K = PER_W // CHUNK
NBUF = 5
LAG = 0


def _emb_body(table_hbm, idx_hbm, out_hbm, idx_v, rows_v, *sems):
    gsems = sems[:NBUF]
    osems = sems[NBUF:]
    wid = lax.axis_index("s") * NC + lax.axis_index("c")

    pltpu.sync_copy(idx_hbm.at[wid], idx_v)

    def start_gather(t, b):
        pltpu.async_copy(table_hbm.at[idx_v.at[t]], rows_v.at[b], gsems[b])

    def wait_gather(t, b):
        pltpu.make_async_copy(
            table_hbm.at[idx_v.at[t]], rows_v.at[b], gsems[b]).wait()

    def start_write(t, b):
        pltpu.async_copy(rows_v.at[b], out_hbm.at[wid * K + t], osems[b])

    def wait_write(t, b):
        pltpu.make_async_copy(
            rows_v.at[b], out_hbm.at[wid * K + t], osems[b]).wait()

    for b in range(NBUF):
        start_gather(b, b)

    for t in range(LAG):
        wait_gather(t, t % NBUF)
        start_write(t, t % NBUF)

    @pl.loop(LAG, K - NBUF + LAG, step=NBUF)
    def _main(t0):
        for i in range(NBUF):
            t = t0 + i
            b = (LAG + i) % NBUF
            bl = i % NBUF
            wait_gather(t, b)
            start_write(t, b)
            wait_write(t - LAG, bl)
            start_gather(t - LAG + NBUF, bl)

    for t in range(K - NBUF + LAG, K):
        wait_gather(t, t % NBUF)
        start_write(t, t % NBUF)
    for t in range(K - NBUF, K):
        wait_write(t, t % NBUF)


@jax.jit
def kernel(data, ivectors):
    idx = jnp.transpose(data.astype(jnp.int32)).reshape(NW, K, CHUNK)
    mesh = plsc.VectorSubcoreMesh(core_axis_name="c", subcore_axis_name="s")
    run = pl.kernel(
        _emb_body,
        out_type=jax.ShapeDtypeStruct((NW * K, CHUNK, FACTORS), jnp.float32),
        mesh=mesh,
        scratch_types=(
            [pltpu.VMEM((K, CHUNK), jnp.int32),
             pltpu.VMEM((NBUF, CHUNK, FACTORS), jnp.float32)]
            + [pltpu.SemaphoreType.DMA] * (2 * NBUF)
        ),
    )
    out = run(ivectors, idx)
    return jnp.transpose(out.reshape(HIST, BATCH, FACTORS), (1, 0, 2))
